# baseline (speedup 1.0000x reference)
import functools

import jax
import jax.numpy as jnp
from jax import lax
from jax.experimental import pallas as pl
from jax.experimental.pallas import tpu as pltpu
from jax.experimental.pallas import tpu_sc as plsc

_NC = 2
_NS = 16
_NW = _NC * _NS

_BR = 512
_A_BLOCKS = 2


def _sc_gather_body(sa_tab_hbm, som_tab_hbm, t_hbm, coef_out_hbm,
                    t_v, sa_o_v, som_o_v, sem_a, sem_b, *, b, b_per_w):
    wid = lax.axis_index("s") * _NC + lax.axis_index("c")
    base = wid * b_per_w
    pltpu.sync_copy(t_hbm.at[pl.ds(base, b_per_w)], t_v)
    cp_a = pltpu.async_copy(sa_tab_hbm.at[t_v], sa_o_v, sem_a)
    cp_b = pltpu.async_copy(som_tab_hbm.at[t_v], som_o_v, sem_b)
    cp_a.wait()
    cp_b.wait()
    pltpu.sync_copy(sa_o_v, coef_out_hbm.at[pl.ds(base, b_per_w)])
    pltpu.sync_copy(som_o_v, coef_out_hbm.at[pl.ds(b + base, b_per_w)])


def _sc_gather(sa_tab, som_tab, t):
    b = t.shape[0]
    b_per_w = b // _NW
    mesh = plsc.VectorSubcoreMesh(core_axis_name="c", subcore_axis_name="s")
    body = functools.partial(_sc_gather_body, b=b, b_per_w=b_per_w)
    k = pl.kernel(
        body,
        out_type=jax.ShapeDtypeStruct((2 * b,), jnp.float32),
        mesh=mesh,
        scratch_types=[
            pltpu.VMEM((b_per_w,), jnp.int32),
            pltpu.VMEM((b_per_w,), jnp.float32),
            pltpu.VMEM((b_per_w,), jnp.float32),
            pltpu.SemaphoreType.DMA,
            pltpu.SemaphoreType.DMA,
        ],
    )
    return k(sa_tab, som_tab, t)


def _onehot_body(t_ref, sa_tab_ref, som_tab_ref, x_ref, n_ref, o_ref):
    br = t_ref.shape[0]
    tt = sa_tab_ref.shape[1]
    tcol = t_ref[...][:, None]
    iota = lax.broadcasted_iota(jnp.int32, (br, tt), 1)
    onehot = (iota == tcol)
    sa = jnp.sum(jnp.where(onehot, sa_tab_ref[...], 0.0), axis=1)[:, None, None]
    som = jnp.sum(jnp.where(onehot, som_tab_ref[...], 0.0), axis=1)[:, None, None]
    o_ref[...] = sa * x_ref[...] + som * n_ref[...]


def _combine_a(t, sa_tab2, som_tab2, x, n):
    b, c, l = x.shape
    tt = sa_tab2.shape[1]
    row_spec = pl.BlockSpec((_BR, c, l), lambda i: (i, 0, 0))
    t_spec = pl.BlockSpec((_BR,), lambda i: (i,))
    tab_spec = pl.BlockSpec((1, tt), lambda i: (0, 0))
    return pl.pallas_call(
        _onehot_body,
        grid=(_A_BLOCKS,),
        in_specs=[t_spec, tab_spec, tab_spec, row_spec, row_spec],
        out_specs=row_spec,
        out_shape=jax.ShapeDtypeStruct((b, c, l), jnp.float32),
    )(t, sa_tab2, som_tab2, x, n)


def _coef_body(sa_ref, som_ref, x_ref, n_ref, prev_ref, o_ref):
    del prev_ref
    sa = sa_ref[...][:, None, None]
    som = som_ref[...][:, None, None]
    o_ref[...] = sa * x_ref[...] + som * n_ref[...]


def _combine_b(coef, x, n, prev):
    b, c, l = x.shape
    nblk = b // _BR
    row_spec = pl.BlockSpec((_BR, c, l), lambda i: (i + _A_BLOCKS, 0, 0))
    sa_spec = pl.BlockSpec((_BR,), lambda i: (i + _A_BLOCKS,))
    som_spec = pl.BlockSpec((_BR,), lambda i: (i + _A_BLOCKS + nblk,))
    return pl.pallas_call(
        _coef_body,
        grid=(nblk - _A_BLOCKS,),
        in_specs=[sa_spec, som_spec, row_spec, row_spec, row_spec],
        out_specs=row_spec,
        out_shape=jax.ShapeDtypeStruct((b, c, l), jnp.float32),
        input_output_aliases={4: 0},
    )(coef, coef, x, n, prev)


def kernel(x_0, t, noise, sqrt_alphas_cumprod, sqrt_one_minus_alphas_cumprod):
    tt = sqrt_alphas_cumprod.shape[0]
    coef = _sc_gather(sqrt_alphas_cumprod, sqrt_one_minus_alphas_cumprod, t)
    out_a = _combine_a(t, sqrt_alphas_cumprod.reshape(1, tt),
                       sqrt_one_minus_alphas_cumprod.reshape(1, tt),
                       x_0, noise)
    return _combine_b(coef, x_0, noise, out_a)


# speedup vs baseline: 1.1521x; 1.1521x over previous
import functools

import jax
import jax.numpy as jnp
from jax import lax
from jax.experimental import pallas as pl
from jax.experimental.pallas import tpu as pltpu
from jax.experimental.pallas import tpu_sc as plsc

_NC = 2
_NS = 16
_NW = _NC * _NS

_BR = 512
_A_BLOCKS = 2


def _sc_gather_body(sa_tab_hbm, som_tab_hbm, t_hbm, coef_out_hbm,
                    t_v, sa_o_v, som_o_v, sem_a, sem_b, *, b, b_per_w):
    wid = lax.axis_index("s") * _NC + lax.axis_index("c")
    base = wid * b_per_w
    pltpu.sync_copy(t_hbm.at[pl.ds(base, b_per_w)], t_v)
    cp_a = pltpu.async_copy(sa_tab_hbm.at[t_v], sa_o_v, sem_a)
    cp_b = pltpu.async_copy(som_tab_hbm.at[t_v], som_o_v, sem_b)
    cp_a.wait()
    cp_b.wait()
    pltpu.sync_copy(sa_o_v, coef_out_hbm.at[pl.ds(base, b_per_w)])
    pltpu.sync_copy(som_o_v, coef_out_hbm.at[pl.ds(b + base, b_per_w)])


def _sc_gather(sa_tab, som_tab, t):
    b = t.shape[0]
    b_per_w = b // _NW
    mesh = plsc.VectorSubcoreMesh(core_axis_name="c", subcore_axis_name="s")
    body = functools.partial(_sc_gather_body, b=b, b_per_w=b_per_w)
    k = pl.kernel(
        body,
        out_type=jax.ShapeDtypeStruct((2 * b,), jnp.float32),
        mesh=mesh,
        scratch_types=[
            pltpu.VMEM((b_per_w,), jnp.int32),
            pltpu.VMEM((b_per_w,), jnp.float32),
            pltpu.VMEM((b_per_w,), jnp.float32),
            pltpu.SemaphoreType.DMA,
            pltpu.SemaphoreType.DMA,
        ],
    )
    return k(sa_tab, som_tab, t)


def _onehot_body(t_ref, sa_tab_ref, som_tab_ref, x_ref, n_ref, o_ref):
    br = t_ref.shape[0]
    tt = sa_tab_ref.shape[1]
    tcol = t_ref[...][:, None]
    iota = lax.broadcasted_iota(jnp.int32, (br, tt), 1)
    onehot = (iota == tcol)
    sa = jnp.sum(jnp.where(onehot, sa_tab_ref[...], 0.0), axis=1)[:, None, None]
    som = jnp.sum(jnp.where(onehot, som_tab_ref[...], 0.0), axis=1)[:, None, None]
    o_ref[...] = sa * x_ref[...] + som * n_ref[...]


def _combine_a(t, sa_tab2, som_tab2, x, n):
    b, c, l = x.shape
    tt = sa_tab2.shape[1]
    row_spec = pl.BlockSpec((_BR, c, l), lambda i: (i, 0, 0))
    t_spec = pl.BlockSpec((_BR,), lambda i: (i,))
    tab_spec = pl.BlockSpec((1, tt), lambda i: (0, 0))
    return pl.pallas_call(
        _onehot_body,
        grid=(_A_BLOCKS,),
        in_specs=[t_spec, tab_spec, tab_spec, row_spec, row_spec],
        out_specs=row_spec,
        out_shape=jax.ShapeDtypeStruct((b, c, l), jnp.float32),
    )(t, sa_tab2, som_tab2, x, n)


def _coef_body(sa_ref, som_ref, x_ref, n_ref, prev_ref, o_ref):
    del prev_ref
    sa = sa_ref[...][:, None, None]
    som = som_ref[...][:, None, None]
    o_ref[...] = sa * x_ref[...] + som * n_ref[...]


def _combine_b(coef, x, n, prev):
    b, c, l = x.shape
    nblk = b // _BR
    row_spec = pl.BlockSpec((_BR, c, l), lambda i: (i + _A_BLOCKS, 0, 0))
    sa_spec = pl.BlockSpec((_BR,), lambda i: (i + _A_BLOCKS,))
    som_spec = pl.BlockSpec((_BR,), lambda i: (i + _A_BLOCKS + nblk,))
    prev_spec = pl.BlockSpec(memory_space=pl.ANY)
    return pl.pallas_call(
        _coef_body,
        grid=(nblk - _A_BLOCKS,),
        in_specs=[sa_spec, som_spec, row_spec, row_spec, prev_spec],
        out_specs=row_spec,
        out_shape=jax.ShapeDtypeStruct((b, c, l), jnp.float32),
        input_output_aliases={4: 0},
    )(coef, coef, x, n, prev)


def kernel(x_0, t, noise, sqrt_alphas_cumprod, sqrt_one_minus_alphas_cumprod):
    tt = sqrt_alphas_cumprod.shape[0]
    coef = _sc_gather(sqrt_alphas_cumprod, sqrt_one_minus_alphas_cumprod, t)
    out_a = _combine_a(t, sqrt_alphas_cumprod.reshape(1, tt),
                       sqrt_one_minus_alphas_cumprod.reshape(1, tt),
                       x_0, noise)
    return _combine_b(coef, x_0, noise, out_a)
